# baseline (speedup 1.0000x reference)
import functools

import jax
import jax.numpy as jnp
from jax import lax
from jax.experimental import pallas as pl
from jax.experimental.pallas import tpu as pltpu
from jax.experimental.pallas import tpu_sc as plsc

NUM_CORES = 2
NUM_SUBCORES = 16
NW = NUM_CORES * NUM_SUBCORES
LANE = 128
BL = 128


def _build_gather(batch: int, seq: int, d_model: int):
    mesh = plsc.VectorSubcoreMesh(
        core_axis_name="c", subcore_axis_name="s",
        num_cores=NUM_CORES, num_subcores=NUM_SUBCORES)

    @functools.partial(
        pl.kernel,
        out_type=jax.ShapeDtypeStruct((seq, d_model, batch), jnp.float32),
        mesh=mesh,
        scratch_types=[
            pltpu.VMEM((seq, BL), jnp.int32),
            pltpu.VMEM((2, BL, LANE), jnp.float32),
            pltpu.VMEM((2, d_model, BL), jnp.float32),
            pltpu.SemaphoreType.DMA,
            pltpu.SemaphoreType.DMA,
            pltpu.SemaphoreType.DMA,
            pltpu.SemaphoreType.DMA,
        ],
        compiler_params=pltpu.CompilerParams(needs_layout_passes=False),
    )
    def gather_kernel(xt_hbm, tpad_hbm, out_hbm, xv, rbuf, tbuf,
                      g0, g1, s0, s1):
        wid = lax.axis_index("s") * NUM_CORES + lax.axis_index("c")
        b0 = wid * BL
        pltpu.sync_copy(xt_hbm.at[:, pl.ds(b0, BL)], xv)
        gsems = (g0, g1)
        ssems = (s0, s1)
        iota = lax.iota(jnp.int32, 16)

        def gather_desc(s, b):
            return pltpu.make_async_copy(
                tpad_hbm.at[xv.at[s]], rbuf.at[b], gsems[b])

        def store_desc(s, b):
            return pltpu.make_async_copy(
                tbuf.at[b], out_hbm.at[s].at[:, pl.ds(b0, BL)], ssems[b])

        def transpose(b):
            src = rbuf.at[b]
            dst = tbuf.at[b]
            for lb in range(BL // 16):
                rows = iota + (16 * lb)
                for d in range(d_model):
                    cols = jnp.full((16,), d, jnp.int32)
                    dst[d, pl.ds(16 * lb, 16)] = plsc.load_gather(
                        src, [rows, cols])

        gather_desc(0, 0).start()

        @pl.loop(0, seq // 2)
        def _(h):
            s = h * 2
            gather_desc(s + 1, 1).start()
            gather_desc(s, 0).wait()

            @pl.when(h > 0)
            def _():
                store_desc(s, 0).wait()
            transpose(0)
            store_desc(s, 0).start()

            @pl.when(s + 2 < seq)
            def _():
                gather_desc(s + 2, 0).start()

            gather_desc(s + 1, 1).wait()

            @pl.when(h > 0)
            def _():
                store_desc(s + 1, 1).wait()
            transpose(1)
            store_desc(s + 1, 1).start()

            @pl.when(s + 2 >= seq)
            def _():
                store_desc(s, 0).wait()
                store_desc(s + 1, 1).wait()

    return gather_kernel


def kernel(x, embedding_matrix):
    batch, seq = x.shape
    _, d_model = embedding_matrix.shape
    tpad = jnp.pad(embedding_matrix, ((0, 0), (0, LANE - d_model)))
    gather = _build_gather(batch, seq, d_model)
    out_t = gather(x.T, tpad)
    return out_t.transpose(2, 0, 1)
